# baseline (speedup 1.0000x reference)
import jax
import jax.numpy as jnp
from jax import lax
from jax.experimental import pallas as pl
from jax.experimental.pallas import tpu as pltpu
from jax.experimental.pallas import tpu_sc as plsc

_NC = 2
_NS = 16
_NW = _NC * _NS

_D = 64
_CHUNK = 512


def _emb_body(idx_hbm, table_hbm, out_hbm, idx_v, rows_v, sem):
    wid = lax.axis_index("s") * _NC + lax.axis_index("c")
    b_per_w = idx_hbm.shape[0] // _NW
    n_chunks = b_per_w // _CHUNK
    base_w = wid * b_per_w

    def body(i, carry):
        base = base_w + i * _CHUNK
        pltpu.sync_copy(idx_hbm.at[pl.ds(base, _CHUNK)], idx_v)
        pltpu.async_copy(table_hbm.at[idx_v], rows_v, sem).wait()
        pltpu.sync_copy(rows_v, out_hbm.at[pl.ds(base, _CHUNK)])
        return carry

    lax.fori_loop(0, n_chunks, body, 0)


def kernel(input, table):
    B = input.shape[0] * input.shape[1]
    idx = input.reshape(B).astype(jnp.int32)
    mesh = plsc.VectorSubcoreMesh(core_axis_name="c", subcore_axis_name="s")
    f = pl.kernel(
        _emb_body,
        out_type=jax.ShapeDtypeStruct((B, _D), jnp.float32),
        mesh=mesh,
        scratch_types=[
            pltpu.VMEM((_CHUNK,), jnp.int32),
            pltpu.VMEM((_CHUNK, _D), jnp.float32),
            pltpu.SemaphoreType.DMA,
        ],
        compiler_params=pltpu.CompilerParams(use_tc_tiling_on_sc=False),
    )
    out = f(idx, table)
    return out.reshape(input.shape[0], input.shape[1], _D)


# speedup vs baseline: 1.0468x; 1.0468x over previous
import jax
import jax.numpy as jnp
from jax import lax
from jax.experimental import pallas as pl
from jax.experimental.pallas import tpu as pltpu
from jax.experimental.pallas import tpu_sc as plsc

_NC = 2
_NS = 16
_NW = _NC * _NS

_D = 64
_CHUNK = 512
_NBUF = 2


def _emb_body(idx_hbm, table_hbm, out_hbm,
              idx0, idx1, rows0, rows1,
              isem0, isem1, gsem0, gsem1, wsem0, wsem1):
    idx_v = (idx0, idx1)
    rows_v = (rows0, rows1)
    isem = (isem0, isem1)
    gsem = (gsem0, gsem1)
    wsem = (wsem0, wsem1)

    wid = lax.axis_index("s") * _NC + lax.axis_index("c")
    b_per_w = idx_hbm.shape[0] // _NW
    n_chunks = b_per_w // _CHUNK
    n_groups = n_chunks // _NBUF
    base_w = wid * b_per_w

    def chunk_base(j):
        return base_w + j * _CHUNK

    for p in range(_NBUF):
        pltpu.async_copy(idx_hbm.at[pl.ds(chunk_base(p), _CHUNK)],
                         idx_v[p], isem[p])
    for p in range(_NBUF):
        pltpu.make_async_copy(idx_hbm.at[pl.ds(chunk_base(p), _CHUNK)],
                              idx_v[p], isem[p]).wait()
        pltpu.async_copy(table_hbm.at[idx_v[p]], rows_v[p], gsem[p])

    def body(g, carry):
        for p in range(_NBUF):
            jold = (g - 1) * _NBUF + p
            jnew = g * _NBUF + p
            pltpu.make_async_copy(table_hbm.at[idx_v[p]], rows_v[p],
                                  gsem[p]).wait()
            pltpu.async_copy(idx_hbm.at[pl.ds(chunk_base(jnew), _CHUNK)],
                             idx_v[p], isem[p])
            pltpu.async_copy(rows_v[p],
                             out_hbm.at[pl.ds(chunk_base(jold), _CHUNK)],
                             wsem[p])
            pltpu.make_async_copy(rows_v[p],
                                  out_hbm.at[pl.ds(chunk_base(jold), _CHUNK)],
                                  wsem[p]).wait()
            pltpu.make_async_copy(idx_hbm.at[pl.ds(chunk_base(jnew), _CHUNK)],
                                  idx_v[p], isem[p]).wait()
            pltpu.async_copy(table_hbm.at[idx_v[p]], rows_v[p], gsem[p])
        return carry

    lax.fori_loop(1, n_groups, body, 0)

    for p in range(_NBUF):
        jold = (n_groups - 1) * _NBUF + p
        pltpu.make_async_copy(table_hbm.at[idx_v[p]], rows_v[p],
                              gsem[p]).wait()
        pltpu.async_copy(rows_v[p],
                         out_hbm.at[pl.ds(chunk_base(jold), _CHUNK)],
                         wsem[p])
    for p in range(_NBUF):
        jold = (n_groups - 1) * _NBUF + p
        pltpu.make_async_copy(rows_v[p],
                              out_hbm.at[pl.ds(chunk_base(jold), _CHUNK)],
                              wsem[p]).wait()


def kernel(input, table):
    B = input.shape[0] * input.shape[1]
    idx = input.reshape(B).astype(jnp.int32)
    mesh = plsc.VectorSubcoreMesh(core_axis_name="c", subcore_axis_name="s")
    f = pl.kernel(
        _emb_body,
        out_type=jax.ShapeDtypeStruct((B, _D), jnp.float32),
        mesh=mesh,
        scratch_types=[
            pltpu.VMEM((_CHUNK,), jnp.int32),
            pltpu.VMEM((_CHUNK,), jnp.int32),
            pltpu.VMEM((_CHUNK, _D), jnp.float32),
            pltpu.VMEM((_CHUNK, _D), jnp.float32),
            pltpu.SemaphoreType.DMA,
            pltpu.SemaphoreType.DMA,
            pltpu.SemaphoreType.DMA,
            pltpu.SemaphoreType.DMA,
            pltpu.SemaphoreType.DMA,
            pltpu.SemaphoreType.DMA,
        ],
        compiler_params=pltpu.CompilerParams(use_tc_tiling_on_sc=False),
    )
    out = f(idx, table)
    return out.reshape(input.shape[0], input.shape[1], _D)
